# baseline (device time: 197951 ns/iter reference)
import jax
import jax.numpy as jnp
from jax import lax
from jax.experimental import pallas as pl
from jax.experimental.pallas import tpu as pltpu

N_DEV = 4
N_HOP = N_DEV - 1


def _ag_gemm(x, w_mat):
    m_per, k = x.shape
    _, n_per = w_mat.shape
    m_half = m_per // 2
    m_q = m_half // 2

    def body(x_hbm, w_hbm, y_hbm,
             comm_a, comm_b, x_bf16, ybuf, w_bf16, stage,
             send_a_sems, recv_a_sems, send_b_sems, recv_b_sems,
             copy_sems, scopy_sem, qout_sems, credit_sems,
             asend_sems, arecv_sems, abuf_send, abuf_recv):
        my = lax.axis_index("i")
        right = lax.rem(my + 1, N_DEV)
        left = lax.rem(my + N_DEV - 1, N_DEV)

        barrier = pltpu.get_barrier_semaphore()
        for nbr in (left, right):
            pl.semaphore_signal(barrier, inc=1, device_id=(nbr,),
                                device_id_type=pl.DeviceIdType.MESH)
        pl.semaphore_wait(barrier, 2)

        def quarter(dirn, q):
            return (0 if dirn == 0 else m_half) + q * m_q

        def ring_desc(dirn, h, q, src_ref):
            comm = comm_a if dirn == 0 else comm_b
            return pltpu.make_async_remote_copy(
                src_ref=src_ref,
                dst_ref=comm.at[h % 2, pl.ds(q * m_q, m_q), :],
                send_sem=(send_a_sems if dirn == 0 else send_b_sems).at[h, q],
                recv_sem=(recv_a_sems if dirn == 0 else recv_b_sems).at[h, q],
                device_id=(right if dirn == 0 else left,),
                device_id_type=pl.DeviceIdType.MESH,
            )

        hop0 = {}
        for i, (dirn, q) in enumerate(((0, 0), (1, 0), (0, 1), (1, 1))):
            row0 = quarter(dirn, q)
            cp = pltpu.make_async_copy(
                x_hbm.at[pl.ds(row0, m_q), :], stage, scopy_sem)
            cp.start()
            cp.wait()
            x_bf16[pl.ds(row0, m_q), :] = stage[...].astype(jnp.bfloat16)
            d = ring_desc(dirn, 0, q, x_bf16.at[pl.ds(row0, m_q), :])
            d.start()
            hop0[(dirn, q)] = d

        for b in range(k // m_q):
            cp = pltpu.make_async_copy(
                w_hbm.at[pl.ds(b * m_q, m_q), :],
                stage.at[:, pl.ds(0, n_per)], scopy_sem)
            cp.start()
            cp.wait()
            w_bf16[pl.ds(b * m_q, m_q), :] = (
                stage[:, pl.ds(0, n_per)].astype(jnp.bfloat16))

        pending = [None, None]
        gemm_n = [0]

        def do_gemm(src_ref, out_row, amax, nrows):
            s = gemm_n[0] % 2
            gemm_n[0] += 1
            y = jnp.dot(src_ref[...], w_bf16[...],
                        preferred_element_type=jnp.float32)
            y = jnp.maximum(y, 0.0)
            amax = jnp.maximum(amax, jnp.max(y))
            if pending[s] is not None:
                pending[s].wait()
            ybuf[s, pl.ds(0, nrows), :] = y.astype(jnp.bfloat16)
            cp = pltpu.make_async_copy(
                ybuf.at[s, pl.ds(0, nrows), :],
                y_hbm.at[pl.ds(out_row, nrows), :],
                copy_sems.at[s])
            cp.start()
            pending[s] = cp
            return amax

        amax = jnp.float32(0.0)
        amax = do_gemm(x_bf16.at[pl.ds(0, m_half), :], my * m_per,
                       amax, m_half)
        amax = do_gemm(x_bf16.at[pl.ds(m_half, m_half), :],
                       my * m_per + m_half, amax, m_half)

        fwd = {}
        for h in range(N_HOP):
            slot = h % 2
            origin_a = lax.rem(my + N_DEV - 1 - h, N_DEV)
            origin_b = lax.rem(my + 1 + h, N_DEV)
            for q in range(2):
                for dirn in range(2):
                    comm = comm_a if dirn == 0 else comm_b
                    ring_desc(dirn, h, q,
                              x_bf16.at[pl.ds(0, m_q), :]).wait_recv()
                    if h == 0:
                        d = ring_desc(
                            dirn, 1, q,
                            comm.at[slot, pl.ds(q * m_q, m_q), :])
                        d.start()
                        fwd[(dirn, 1, q)] = d
                    if h == 2:
                        origin = origin_a if dirn == 0 else origin_b
                        row = (origin * m_per + quarter(dirn, 0) + q * m_q)
                        amax = do_gemm(
                            comm.at[slot, pl.ds(q * m_q, m_q), :],
                            row, amax, m_q)
            if h == 1:
                for key in ((0, 1, 0), (0, 1, 1), (1, 1, 0), (1, 1, 1)):
                    fwd[key].wait_send()
                pl.semaphore_signal(credit_sems.at[0], inc=1,
                                    device_id=(left,),
                                    device_id_type=pl.DeviceIdType.MESH)
                pl.semaphore_signal(credit_sems.at[1], inc=1,
                                    device_id=(right,),
                                    device_id_type=pl.DeviceIdType.MESH)
                pl.semaphore_wait(credit_sems.at[0], 1)
                pl.semaphore_wait(credit_sems.at[1], 1)
                for q in range(2):
                    for dirn in range(2):
                        comm = comm_a if dirn == 0 else comm_b
                        d = ring_desc(
                            dirn, 2, q,
                            comm.at[slot, pl.ds(q * m_q, m_q), :])
                        d.start()
                        fwd[(dirn, 2, q)] = d
            if h < 2:
                amax = do_gemm(comm_a.at[slot], origin_a * m_per,
                               amax, m_half)
                amax = do_gemm(comm_b.at[slot],
                               origin_b * m_per + m_half, amax, m_half)

        for d in hop0.values():
            d.wait_send()
        for key in ((0, 2, 0), (0, 2, 1), (1, 2, 0), (1, 2, 1)):
            fwd[key].wait_send()

        abuf_send[...] = jnp.full((8, 128), amax, jnp.float32)
        sends = []
        for j in range(N_DEV - 1):
            tgt = lax.rem(my + 1 + j, N_DEV)
            slot = N_DEV - 2 - j
            d = pltpu.make_async_remote_copy(
                src_ref=abuf_send,
                dst_ref=abuf_recv.at[slot],
                send_sem=asend_sems.at[j],
                recv_sem=arecv_sems.at[slot],
                device_id=(tgt,),
                device_id_type=pl.DeviceIdType.MESH,
            )
            d.start()
            sends.append(d)
        for slot in range(N_DEV - 1):
            pltpu.make_async_remote_copy(
                src_ref=abuf_send,
                dst_ref=abuf_recv.at[slot],
                send_sem=asend_sems.at[0],
                recv_sem=arecv_sems.at[slot],
                device_id=(left,),
                device_id_type=pl.DeviceIdType.MESH,
            ).wait_recv()
        for d in sends:
            d.wait_send()
        for p in pending:
            if p is not None:
                p.wait()

        for slot in range(N_DEV - 1):
            amax = jnp.maximum(amax, abuf_recv[slot, 0, 0])

        inv = 127.0 / amax
        scale = amax / 127.0
        n_blk = (N_DEV * m_per) // m_half
        qin = [None, None]
        qout = [None, None]

        def start_in(b):
            s = b % 2
            cp = pltpu.make_async_copy(
                y_hbm.at[pl.ds(b * m_half, m_half), :], ybuf.at[s],
                copy_sems.at[s])
            cp.start()
            qin[s] = cp

        start_in(0)
        for b in range(n_blk):
            s = b % 2
            qin[s].wait()
            if b + 1 < n_blk:
                if qout[(b + 1) % 2] is not None:
                    qout[(b + 1) % 2].wait()
                start_in(b + 1)
            y = ybuf[s].astype(jnp.float32)
            q = jnp.clip(jnp.round(y * inv), 0.0, 127.0)
            ybuf[s] = (q * scale).astype(jnp.bfloat16)
            cp = pltpu.make_async_copy(
                ybuf.at[s], y_hbm.at[pl.ds(b * m_half, m_half), :],
                qout_sems.at[s])
            cp.start()
            qout[s] = cp
        for p in qout:
            if p is not None:
                p.wait()

    return pl.pallas_call(
        body,
        out_shape=jax.ShapeDtypeStruct((N_DEV * m_per, n_per),
                                       jnp.bfloat16),
        in_specs=[
            pl.BlockSpec(memory_space=pl.ANY),
            pl.BlockSpec(memory_space=pl.ANY),
        ],
        out_specs=pl.BlockSpec(memory_space=pl.ANY),
        scratch_shapes=[
            pltpu.VMEM((2, m_half, k), jnp.bfloat16),
            pltpu.VMEM((2, m_half, k), jnp.bfloat16),
            pltpu.VMEM((m_per, k), jnp.bfloat16),
            pltpu.VMEM((2, m_half, n_per), jnp.bfloat16),
            pltpu.VMEM((k, n_per), jnp.bfloat16),
            pltpu.VMEM((m_q, k), jnp.float32),
            pltpu.SemaphoreType.DMA((N_HOP, 2)),
            pltpu.SemaphoreType.DMA((N_HOP, 2)),
            pltpu.SemaphoreType.DMA((N_HOP, 2)),
            pltpu.SemaphoreType.DMA((N_HOP, 2)),
            pltpu.SemaphoreType.DMA((2,)),
            pltpu.SemaphoreType.DMA,
            pltpu.SemaphoreType.DMA((2,)),
            pltpu.SemaphoreType.REGULAR((2,)),
            pltpu.SemaphoreType.DMA((N_DEV - 1,)),
            pltpu.SemaphoreType.DMA((N_DEV - 1,)),
            pltpu.VMEM((8, 128), jnp.float32),
            pltpu.VMEM((N_DEV - 1, 8, 128), jnp.float32),
        ],
        compiler_params=pltpu.CompilerParams(
            collective_id=0, vmem_limit_bytes=100 * 1024 * 1024),
    )(x, w_mat)


def kernel(x, w_mat):
    return _ag_gemm(x, w_mat)


# device time: 188126 ns/iter; 1.0522x vs baseline; 1.0522x over previous
import jax
import jax.numpy as jnp
from jax import lax
from jax.experimental import pallas as pl
from jax.experimental.pallas import tpu as pltpu

N_DEV = 4
N_HOP = N_DEV - 1


def _ag_gemm(x, w_mat):
    m_per, k = x.shape
    _, n_per = w_mat.shape
    m_half = m_per // 2
    m_q = m_half // 2

    def body(x_hbm, w_hbm, y_hbm, amax_ref,
             comm_a, comm_b, x_bf16, ybuf, w_bf16, stage,
             send_a_sems, recv_a_sems, send_b_sems, recv_b_sems,
             copy_sem, scopy_sem, credit_sems, asend_sems, arecv_sems,
             abuf_send, abuf_recv):
        my = lax.axis_index("i")
        right = lax.rem(my + 1, N_DEV)
        left = lax.rem(my + N_DEV - 1, N_DEV)

        barrier = pltpu.get_barrier_semaphore()
        for nbr in (left, right):
            pl.semaphore_signal(barrier, inc=1, device_id=(nbr,),
                                device_id_type=pl.DeviceIdType.MESH)
        pl.semaphore_wait(barrier, 2)

        def quarter(dirn, q):
            return (0 if dirn == 0 else m_half) + q * m_q

        def ring_desc(dirn, h, q, src_ref):
            comm = comm_a if dirn == 0 else comm_b
            return pltpu.make_async_remote_copy(
                src_ref=src_ref,
                dst_ref=comm.at[h % 2, pl.ds(q * m_q, m_q), :],
                send_sem=(send_a_sems if dirn == 0 else send_b_sems).at[h, q],
                recv_sem=(recv_a_sems if dirn == 0 else recv_b_sems).at[h, q],
                device_id=(right if dirn == 0 else left,),
                device_id_type=pl.DeviceIdType.MESH,
            )

        hop0 = {}
        for i, (dirn, q) in enumerate(((0, 0), (1, 0), (0, 1), (1, 1))):
            row0 = quarter(dirn, q)
            cp = pltpu.make_async_copy(
                x_hbm.at[pl.ds(row0, m_q), :], stage, scopy_sem)
            cp.start()
            cp.wait()
            x_bf16[pl.ds(row0, m_q), :] = stage[...].astype(jnp.bfloat16)
            d = ring_desc(dirn, 0, q, x_bf16.at[pl.ds(row0, m_q), :])
            d.start()
            hop0[(dirn, q)] = d

        for b in range(k // m_q):
            cp = pltpu.make_async_copy(
                w_hbm.at[pl.ds(b * m_q, m_q), :],
                stage.at[:, pl.ds(0, n_per)], scopy_sem)
            cp.start()
            cp.wait()
            w_bf16[pl.ds(b * m_q, m_q), :] = (
                stage[:, pl.ds(0, n_per)].astype(jnp.bfloat16))

        pending = [None]

        def do_gemm(src_ref, out_row, amax, nrows):
            y = jnp.dot(src_ref[...], w_bf16[...],
                        preferred_element_type=jnp.float32)
            y = jnp.maximum(y, 0.0)
            amax = jnp.maximum(amax, jnp.max(y))
            if pending[0] is not None:
                pending[0].wait()
            ybuf[pl.ds(0, nrows), :] = y.astype(jnp.bfloat16)
            cp = pltpu.make_async_copy(
                ybuf.at[pl.ds(0, nrows), :],
                y_hbm.at[pl.ds(out_row, nrows), :],
                copy_sem)
            cp.start()
            pending[0] = cp
            return amax

        amax = jnp.float32(0.0)
        amax = do_gemm(x_bf16.at[pl.ds(0, m_half), :], my * m_per,
                       amax, m_half)
        amax = do_gemm(x_bf16.at[pl.ds(m_half, m_half), :],
                       my * m_per + m_half, amax, m_half)

        fwd = {}
        for h in range(N_HOP):
            slot = h % 2
            origin_a = lax.rem(my + N_DEV - 1 - h, N_DEV)
            origin_b = lax.rem(my + 1 + h, N_DEV)
            for q in range(2):
                for dirn in range(2):
                    comm = comm_a if dirn == 0 else comm_b
                    ring_desc(dirn, h, q,
                              x_bf16.at[pl.ds(0, m_q), :]).wait_recv()
                    if h == 0:
                        d = ring_desc(
                            dirn, 1, q,
                            comm.at[slot, pl.ds(q * m_q, m_q), :])
                        d.start()
                        fwd[(dirn, 1, q)] = d
                    if h == 2:
                        origin = origin_a if dirn == 0 else origin_b
                        row = (origin * m_per + quarter(dirn, 0) + q * m_q)
                        amax = do_gemm(
                            comm.at[slot, pl.ds(q * m_q, m_q), :],
                            row, amax, m_q)
            if h == 1:
                for key in ((0, 1, 0), (0, 1, 1), (1, 1, 0), (1, 1, 1)):
                    fwd[key].wait_send()
                pl.semaphore_signal(credit_sems.at[0], inc=1,
                                    device_id=(left,),
                                    device_id_type=pl.DeviceIdType.MESH)
                pl.semaphore_signal(credit_sems.at[1], inc=1,
                                    device_id=(right,),
                                    device_id_type=pl.DeviceIdType.MESH)
                pl.semaphore_wait(credit_sems.at[0], 1)
                pl.semaphore_wait(credit_sems.at[1], 1)
                for q in range(2):
                    for dirn in range(2):
                        comm = comm_a if dirn == 0 else comm_b
                        d = ring_desc(
                            dirn, 2, q,
                            comm.at[slot, pl.ds(q * m_q, m_q), :])
                        d.start()
                        fwd[(dirn, 2, q)] = d
            if h < 2:
                amax = do_gemm(comm_a.at[slot], origin_a * m_per,
                               amax, m_half)
                amax = do_gemm(comm_b.at[slot],
                               origin_b * m_per + m_half, amax, m_half)

        for d in hop0.values():
            d.wait_send()
        for key in ((0, 2, 0), (0, 2, 1), (1, 2, 0), (1, 2, 1)):
            fwd[key].wait_send()

        abuf_send[...] = jnp.full((8, 128), amax, jnp.float32)
        sends = []
        for j in range(N_DEV - 1):
            tgt = lax.rem(my + 1 + j, N_DEV)
            slot = N_DEV - 2 - j
            d = pltpu.make_async_remote_copy(
                src_ref=abuf_send,
                dst_ref=abuf_recv.at[slot],
                send_sem=asend_sems.at[j],
                recv_sem=arecv_sems.at[slot],
                device_id=(tgt,),
                device_id_type=pl.DeviceIdType.MESH,
            )
            d.start()
            sends.append(d)
        for slot in range(N_DEV - 1):
            pltpu.make_async_remote_copy(
                src_ref=abuf_send,
                dst_ref=abuf_recv.at[slot],
                send_sem=asend_sems.at[0],
                recv_sem=arecv_sems.at[slot],
                device_id=(left,),
                device_id_type=pl.DeviceIdType.MESH,
            ).wait_recv()
        for d in sends:
            d.wait_send()
        if pending[0] is not None:
            pending[0].wait()

        for slot in range(N_DEV - 1):
            amax = jnp.maximum(amax, abuf_recv[slot, 0, 0])
        amax_ref[...] = jnp.full((8, 128), amax, jnp.float32)

    return pl.pallas_call(
        body,
        out_shape=(
            jax.ShapeDtypeStruct((N_DEV * m_per, n_per), jnp.bfloat16),
            jax.ShapeDtypeStruct((8, 128), jnp.float32),
        ),
        in_specs=[
            pl.BlockSpec(memory_space=pl.ANY),
            pl.BlockSpec(memory_space=pl.ANY),
        ],
        out_specs=(
            pl.BlockSpec(memory_space=pl.ANY),
            pl.BlockSpec(memory_space=pltpu.VMEM),
        ),
        scratch_shapes=[
            pltpu.VMEM((2, m_half, k), jnp.bfloat16),
            pltpu.VMEM((2, m_half, k), jnp.bfloat16),
            pltpu.VMEM((m_per, k), jnp.bfloat16),
            pltpu.VMEM((m_half, n_per), jnp.bfloat16),
            pltpu.VMEM((k, n_per), jnp.bfloat16),
            pltpu.VMEM((m_q, k), jnp.float32),
            pltpu.SemaphoreType.DMA((N_HOP, 2)),
            pltpu.SemaphoreType.DMA((N_HOP, 2)),
            pltpu.SemaphoreType.DMA((N_HOP, 2)),
            pltpu.SemaphoreType.DMA((N_HOP, 2)),
            pltpu.SemaphoreType.DMA,
            pltpu.SemaphoreType.DMA,
            pltpu.SemaphoreType.REGULAR((2,)),
            pltpu.SemaphoreType.DMA((N_DEV - 1,)),
            pltpu.SemaphoreType.DMA((N_DEV - 1,)),
            pltpu.VMEM((8, 128), jnp.float32),
            pltpu.VMEM((N_DEV - 1, 8, 128), jnp.float32),
        ],
        compiler_params=pltpu.CompilerParams(
            collective_id=0, vmem_limit_bytes=100 * 1024 * 1024),
    )(x, w_mat)


def _quantize(y_pre, amax_tile):
    m, n = y_pre.shape
    blk = 512

    def body(y_ref, amax_ref, o_ref):
        amax = amax_ref[0, 0]
        scale = amax / 127.0
        y = y_ref[...].astype(jnp.float32)
        q = jnp.clip(jnp.round(y * (127.0 / amax)), 0.0, 127.0)
        o_ref[...] = (q * scale).astype(jnp.bfloat16)

    return pl.pallas_call(
        body,
        grid=(m // blk,),
        out_shape=jax.ShapeDtypeStruct((m, n), jnp.bfloat16),
        in_specs=[
            pl.BlockSpec((blk, n), lambda i: (i, 0)),
            pl.BlockSpec((8, 128), lambda i: (0, 0)),
        ],
        out_specs=pl.BlockSpec((blk, n), lambda i: (i, 0)),
    )(y_pre, amax_tile)


def kernel(x, w_mat):
    y_pre, amax_tile = _ag_gemm(x, w_mat)
    return _quantize(y_pre, amax_tile)
